# baseline (speedup 1.0000x reference)
import functools

import jax
import jax.numpy as jnp
from jax import lax
from jax.experimental import pallas as pl
from jax.experimental.pallas import tpu as pltpu
from jax.experimental.pallas import tpu_sc as plsc

B, C, H, W = 256, 1, 512, 64
NODE = 4
H_OUT = H * NODE

NUM_WORKERS = 32
W_STRIP = 16
STRIPS_PER_BATCH = W // W_STRIP
NUM_STRIPS = B * STRIPS_PER_BATCH
STRIPS_PER_WORKER = NUM_STRIPS // NUM_WORKERS
LANES = 16
SEGS = H // LANES
OUT_SEGS = H_OUT // LANES


def kernel(hidden, indices, node_count):
    del node_count
    hid_t = jnp.transpose(hidden, (0, 1, 3, 2))
    idx_t = jnp.transpose(indices, (0, 1, 3, 2))

    mesh = plsc.VectorSubcoreMesh(core_axis_name="c", subcore_axis_name="s")

    @functools.partial(
        pl.kernel,
        out_type=jax.ShapeDtypeStruct((B, W, H_OUT), jnp.float32),
        mesh=mesh,
        compiler_params=pltpu.CompilerParams(
            needs_layout_passes=False, use_tc_tiling_on_sc=True),
        scratch_types=[
            pltpu.VMEM((W_STRIP, H), jnp.int32),
            pltpu.VMEM((W_STRIP, H), jnp.int32),
            pltpu.VMEM((W_STRIP, H), jnp.float32),
            pltpu.VMEM((W_STRIP, H), jnp.float32),
            pltpu.VMEM((W_STRIP, H_OUT), jnp.float32),
            pltpu.VMEM((W_STRIP, H_OUT), jnp.float32),
            pltpu.SemaphoreType.DMA,
            pltpu.SemaphoreType.DMA,
            pltpu.SemaphoreType.DMA,
            pltpu.SemaphoreType.DMA,
        ],
    )
    def run(hid_hbm, idx_hbm, out_hbm, idxbuf0, idxbuf1, hidbuf0, hidbuf1,
            outbuf0, outbuf1, insem0, insem1, outsem0, outsem1):
        idxbufs = (idxbuf0, idxbuf1)
        hidbufs = (hidbuf0, hidbuf1)
        outbufs = (outbuf0, outbuf1)
        insems = (insem0, insem1)
        outsems = (outsem0, outsem1)
        wid = lax.axis_index("s") * 2 + lax.axis_index("c")
        c0 = wid * STRIPS_PER_WORKER
        zeros = jnp.zeros((LANES,), jnp.float32)

        def in_slices(c):
            b = lax.div(c, STRIPS_PER_BATCH)
            w0 = lax.rem(c, STRIPS_PER_BATCH) * W_STRIP
            return (idx_hbm.at[b, 0, pl.ds(w0, W_STRIP), :],
                    hid_hbm.at[b, 0, pl.ds(w0, W_STRIP), :])

        def out_slice(c):
            b = lax.div(c, STRIPS_PER_BATCH)
            w0 = lax.rem(c, STRIPS_PER_BATCH) * W_STRIP
            return out_hbm.at[b, pl.ds(w0, W_STRIP), :]

        def start_in(c, s):
            isl, hsl = in_slices(c)
            pltpu.async_copy(isl, idxbufs[s], insems[s])
            pltpu.async_copy(hsl, hidbufs[s], insems[s])

        def wait_in(c, s):
            isl, hsl = in_slices(c)
            pltpu.make_async_copy(isl, idxbufs[s], insems[s]).wait()
            pltpu.make_async_copy(hsl, hidbufs[s], insems[s]).wait()

        start_in(c0, 0)
        start_in(c0 + 1, 1)

        def pair_body(t2, carry):
            for s in range(2):
                t = t2 * 2 + s
                c = c0 + t
                outbuf = outbufs[s]

                @pl.when(t >= 2)
                def _wait_prev_out():
                    pltpu.make_async_copy(outbuf, out_slice(c), outsems[s]).wait()

                def zero_body(wr, carry2):
                    for seg in range(OUT_SEGS):
                        outbuf[wr, pl.ds(seg * LANES, LANES)] = zeros
                    return carry2

                lax.fori_loop(0, W_STRIP, zero_body, 0, unroll=4)
                wait_in(c, s)
                idxbuf, hidbuf = idxbufs[s], hidbufs[s]

                def scat_body(wr, carry2):
                    row = jnp.full((LANES,), 0, jnp.int32) + wr
                    for seg in range(SEGS):
                        iv = idxbuf[wr, pl.ds(seg * LANES, LANES)]
                        col = lax.shift_right_logical(iv, 6)
                        vv = hidbuf[wr, pl.ds(seg * LANES, LANES)]
                        plsc.store_scatter(outbuf, [row, col], vv)
                    return carry2

                lax.fori_loop(0, W_STRIP, scat_body, 0, unroll=4)
                pltpu.async_copy(outbuf, out_slice(c), outsems[s])

                @pl.when(t + 2 < STRIPS_PER_WORKER)
                def _prefetch_in():
                    start_in(c + 2, s)
            return carry

        lax.fori_loop(0, STRIPS_PER_WORKER // 2, pair_body, 0)
        for s in range(2):
            c_last = c0 + STRIPS_PER_WORKER - 2 + s
            pltpu.make_async_copy(outbufs[s], out_slice(c_last), outsems[s]).wait()

    out_t = run(hid_t, idx_t)
    return jnp.transpose(out_t, (0, 2, 1))


# speedup vs baseline: 1.0585x; 1.0585x over previous
import functools

import jax
import jax.numpy as jnp
from jax import lax
from jax.experimental import pallas as pl
from jax.experimental.pallas import tpu as pltpu
from jax.experimental.pallas import tpu_sc as plsc

B, C, H, W = 256, 1, 512, 64
NODE = 4
H_OUT = H * NODE

NUM_WORKERS = 32
W_STRIP = 16
STRIPS_PER_BATCH = W // W_STRIP
NUM_STRIPS = B * STRIPS_PER_BATCH
STRIPS_PER_WORKER = NUM_STRIPS // NUM_WORKERS
LANES = 16
SEGS = H // LANES
OUT_SEGS = H_OUT // LANES


def kernel(hidden, indices, node_count):
    del node_count
    hid_t = jnp.transpose(hidden, (0, 1, 3, 2))
    idx_t = jnp.transpose(indices, (0, 1, 3, 2))

    mesh = plsc.VectorSubcoreMesh(core_axis_name="c", subcore_axis_name="s")

    @functools.partial(
        pl.kernel,
        out_type=jax.ShapeDtypeStruct((B, W, H_OUT), jnp.float32),
        mesh=mesh,
        compiler_params=pltpu.CompilerParams(
            needs_layout_passes=False, use_tc_tiling_on_sc=True),
        scratch_types=[
            pltpu.VMEM((W_STRIP, H), jnp.int32),
            pltpu.VMEM((W_STRIP, H), jnp.int32),
            pltpu.VMEM((W_STRIP, H), jnp.float32),
            pltpu.VMEM((W_STRIP, H), jnp.float32),
            pltpu.VMEM((W_STRIP, H_OUT), jnp.float32),
            pltpu.VMEM((W_STRIP, H_OUT), jnp.float32),
            pltpu.SemaphoreType.DMA,
            pltpu.SemaphoreType.DMA,
            pltpu.SemaphoreType.DMA,
            pltpu.SemaphoreType.DMA,
        ],
    )
    def run(hid_hbm, idx_hbm, out_hbm, idxbuf0, idxbuf1, hidbuf0, hidbuf1,
            outbuf0, outbuf1, insem0, insem1, outsem0, outsem1):
        idxbufs = (idxbuf0, idxbuf1)
        hidbufs = (hidbuf0, hidbuf1)
        outbufs = (outbuf0, outbuf1)
        insems = (insem0, insem1)
        outsems = (outsem0, outsem1)
        wid = lax.axis_index("s") * 2 + lax.axis_index("c")
        c0 = wid * STRIPS_PER_WORKER
        zeros = jnp.zeros((LANES,), jnp.float32)

        def in_slices(c):
            b = lax.div(c, STRIPS_PER_BATCH)
            w0 = lax.rem(c, STRIPS_PER_BATCH) * W_STRIP
            return (idx_hbm.at[b, 0, pl.ds(w0, W_STRIP), :],
                    hid_hbm.at[b, 0, pl.ds(w0, W_STRIP), :])

        def out_slice(c):
            b = lax.div(c, STRIPS_PER_BATCH)
            w0 = lax.rem(c, STRIPS_PER_BATCH) * W_STRIP
            return out_hbm.at[b, pl.ds(w0, W_STRIP), :]

        def start_in(c, s):
            isl, hsl = in_slices(c)
            pltpu.async_copy(isl, idxbufs[s], insems[s])
            pltpu.async_copy(hsl, hidbufs[s], insems[s])

        def wait_in(c, s):
            isl, hsl = in_slices(c)
            pltpu.make_async_copy(isl, idxbufs[s], insems[s]).wait()
            pltpu.make_async_copy(hsl, hidbufs[s], insems[s]).wait()

        start_in(c0, 0)
        start_in(c0 + 1, 1)

        def pair_body(t2, carry):
            for s in range(2):
                t = t2 * 2 + s
                c = c0 + t
                outbuf = outbufs[s]

                @pl.when(t >= 2)
                def _wait_prev_out():
                    pltpu.make_async_copy(outbuf, out_slice(c), outsems[s]).wait()

                def zero_body(wr, carry2):
                    for seg in range(OUT_SEGS):
                        outbuf[wr, pl.ds(seg * LANES, LANES)] = zeros
                    return carry2

                lax.fori_loop(0, W_STRIP, zero_body, 0, unroll=2)
                wait_in(c, s)
                idxbuf, hidbuf = idxbufs[s], hidbufs[s]

                def scat_body(wr, carry2):
                    row = jnp.full((LANES,), 0, jnp.int32) + wr
                    for seg in range(SEGS):
                        iv = idxbuf[wr, pl.ds(seg * LANES, LANES)]
                        col = lax.shift_right_logical(iv, 6)
                        vv = hidbuf[wr, pl.ds(seg * LANES, LANES)]
                        plsc.store_scatter(outbuf, [row, col], vv)
                    return carry2

                lax.fori_loop(0, W_STRIP, scat_body, 0, unroll=2)
                pltpu.async_copy(outbuf, out_slice(c), outsems[s])

                @pl.when(t + 2 < STRIPS_PER_WORKER)
                def _prefetch_in():
                    start_in(c + 2, s)
            return carry

        lax.fori_loop(0, STRIPS_PER_WORKER // 2, pair_body, 0)
        for s in range(2):
            c_last = c0 + STRIPS_PER_WORKER - 2 + s
            pltpu.make_async_copy(outbufs[s], out_slice(c_last), outsems[s]).wait()

    out_t = run(hid_t, idx_t)
    return jnp.transpose(out_t, (0, 2, 1))
